# Initial kernel scaffold; baseline (speedup 1.0000x reference)
import jax
import jax.numpy as jnp
from jax.experimental import pallas as pl


def kernel(nodes, edges, senders, receivers, params):
    raise NotImplementedError("write your pallas kernel here")


# speedup vs baseline: 1.2088x; 1.2088x over previous
import functools

import jax
import jax.numpy as jnp
from jax import lax
from jax.experimental import pallas as pl
from jax.experimental.pallas import tpu as pltpu

N_NODES = 10000
N_EDGES = 320000
D = 128


def _ln(x):
    mu = jnp.mean(x, axis=-1, keepdims=True)
    var = jnp.mean((x - mu) ** 2, axis=-1, keepdims=True)
    return (x - mu) / jnp.sqrt(var + 1e-6)


def _enc_node_body(x_ref, w1_ref, b1_ref, w2_ref, b2_ref, o_ref):
    x = x_ref[...]
    t = jnp.maximum(jnp.dot(x, w1_ref[...], preferred_element_type=jnp.float32)
                    + b1_ref[...], 0.0)
    y = jnp.dot(t, w2_ref[...], preferred_element_type=jnp.float32) + b2_ref[...]
    o_ref[...] = _ln(y)


def _enc_apply(x, p, block_rows):
    n, din = x.shape
    w1, b1 = p[0]["w"], p[0]["b"].reshape(1, -1)
    w2, b2 = p[1]["w"], p[1]["b"].reshape(1, -1)
    grid = (n // block_rows,)
    return pl.pallas_call(
        _enc_node_body,
        grid=grid,
        in_specs=[
            pl.BlockSpec((block_rows, din), lambda i: (i, 0)),
            pl.BlockSpec(w1.shape, lambda i: (0, 0)),
            pl.BlockSpec(b1.shape, lambda i: (0, 0)),
            pl.BlockSpec(w2.shape, lambda i: (0, 0)),
            pl.BlockSpec(b2.shape, lambda i: (0, 0)),
        ],
        out_specs=pl.BlockSpec((block_rows, D), lambda i: (i, 0)),
        out_shape=jax.ShapeDtypeStruct((n, D), jnp.float32),
    )(x, w1, b1, w2, b2)


def _prep_body(h_ref, ws_ref, wr_ref, a_ref, b_ref):
    h = h_ref[...]
    a_ref[...] = jnp.dot(h, ws_ref[...], preferred_element_type=jnp.float32)
    b_ref[...] = jnp.dot(h, wr_ref[...], preferred_element_type=jnp.float32)


def _prep_tables(h, ws, wr, block_rows=2000):
    grid = (N_NODES // block_rows,)
    return pl.pallas_call(
        _prep_body,
        grid=grid,
        in_specs=[
            pl.BlockSpec((block_rows, D), lambda i: (i, 0)),
            pl.BlockSpec((D, D), lambda i: (0, 0)),
            pl.BlockSpec((D, D), lambda i: (0, 0)),
        ],
        out_specs=[
            pl.BlockSpec((block_rows, D), lambda i: (i, 0)),
            pl.BlockSpec((block_rows, D), lambda i: (i, 0)),
        ],
        out_shape=[
            jax.ShapeDtypeStruct((N_NODES, D), jnp.float32),
            jax.ShapeDtypeStruct((N_NODES, D), jnp.float32),
        ],
    )(h, ws, wr)


def _edge_body(hs_ref, hr_ref, ee_ref, we_ref, b1_ref, w2_ref, b2_ref, o_ref):
    pre = (hs_ref[...] + hr_ref[...]
           + jnp.dot(ee_ref[...], we_ref[...], preferred_element_type=jnp.float32)
           + b1_ref[...])
    t = jnp.maximum(pre, 0.0)
    y = jnp.dot(t, w2_ref[...], preferred_element_type=jnp.float32) + b2_ref[...]
    o_ref[...] = _ln(y)


def _edge_mlp(hs, hr, ee, we, b1, w2, b2, block_rows=4000):
    grid = (N_EDGES // block_rows,)
    b1 = b1.reshape(1, -1)
    b2 = b2.reshape(1, -1)
    return pl.pallas_call(
        _edge_body,
        grid=grid,
        in_specs=[
            pl.BlockSpec((block_rows, D), lambda i: (i, 0)),
            pl.BlockSpec((block_rows, D), lambda i: (i, 0)),
            pl.BlockSpec((block_rows, D), lambda i: (i, 0)),
            pl.BlockSpec((D, D), lambda i: (0, 0)),
            pl.BlockSpec((1, D), lambda i: (0, 0)),
            pl.BlockSpec((D, D), lambda i: (0, 0)),
            pl.BlockSpec((1, D), lambda i: (0, 0)),
        ],
        out_specs=pl.BlockSpec((block_rows, D), lambda i: (i, 0)),
        out_shape=jax.ShapeDtypeStruct((N_EDGES, D), jnp.float32),
    )(hs, hr, ee, we, b1, w2, b2)


def _node_body(h_ref, agg_ref, u1h_ref, u1a_ref, b1_ref, u2_ref, b2_ref, o_ref):
    h = h_ref[...]
    agg = agg_ref[...]
    t = jnp.maximum(
        jnp.dot(h, u1h_ref[...], preferred_element_type=jnp.float32)
        + jnp.dot(agg, u1a_ref[...], preferred_element_type=jnp.float32)
        + b1_ref[...], 0.0)
    y = jnp.dot(t, u2_ref[...], preferred_element_type=jnp.float32) + b2_ref[...]
    o_ref[...] = h + _ln(y)


def _node_mlp(h, agg, u1h, u1a, b1, u2, b2, block_rows=2000):
    grid = (N_NODES // block_rows,)
    b1 = b1.reshape(1, -1)
    b2 = b2.reshape(1, -1)
    return pl.pallas_call(
        _node_body,
        grid=grid,
        in_specs=[
            pl.BlockSpec((block_rows, D), lambda i: (i, 0)),
            pl.BlockSpec((block_rows, D), lambda i: (i, 0)),
            pl.BlockSpec((D, D), lambda i: (0, 0)),
            pl.BlockSpec((D, D), lambda i: (0, 0)),
            pl.BlockSpec((1, D), lambda i: (0, 0)),
            pl.BlockSpec((D, D), lambda i: (0, 0)),
            pl.BlockSpec((1, D), lambda i: (0, 0)),
        ],
        out_specs=pl.BlockSpec((block_rows, D), lambda i: (i, 0)),
        out_shape=jax.ShapeDtypeStruct((N_NODES, D), jnp.float32),
    )(h, agg, u1h, u1a, b1, u2, b2)


def kernel(nodes, edges, senders, receivers, params):
    senders = senders.astype(jnp.int32)
    receivers = receivers.astype(jnp.int32)

    h = _enc_apply(nodes, params["enc_node"], block_rows=2000)
    ee = _enc_apply(edges, params["enc_edge"], block_rows=4000)

    for lp in params["layers"]:
        mw1 = lp["msg"][0]["w"]
        mb1 = lp["msg"][0]["b"]
        mw2, mb2 = lp["msg"][1]["w"], lp["msg"][1]["b"]
        ws, wr, we = mw1[:D], mw1[D:2 * D], mw1[2 * D:]

        a_tab, b_tab = _prep_tables(h, ws, wr)

        hs = a_tab[senders]
        hr = b_tab[receivers]

        msgs = _edge_mlp(hs, hr, ee, we, mb1, mw2, mb2)

        agg = jax.ops.segment_sum(msgs, receivers, num_segments=N_NODES)

        nw1 = lp["node"][0]["w"]
        nb1 = lp["node"][0]["b"]
        nw2, nb2 = lp["node"][1]["w"], lp["node"][1]["b"]
        h = _node_mlp(h, agg, nw1[:D], nw1[D:], nb1, nw2, nb2)

    return h


# speedup vs baseline: 3.3231x; 2.7491x over previous
import functools

import jax
import jax.numpy as jnp
from jax import lax
from jax.experimental import pallas as pl
from jax.experimental.pallas import tpu as pltpu
from jax.experimental.pallas import tpu_sc as plsc

N_NODES = 10000
N_EDGES = 320000
D = 128

_NC = 2
_NS = 16
_NW = _NC * _NS
_EPW = N_EDGES // _NW
_CH = 128
_FULL = _EPW // _CH
_TAIL = _EPW - _FULL * _CH


def _ln(x):
    mu = jnp.mean(x, axis=-1, keepdims=True)
    var = jnp.mean((x - mu) ** 2, axis=-1, keepdims=True)
    return (x - mu) / jnp.sqrt(var + 1e-6)


def _enc_node_body(x_ref, w1_ref, b1_ref, w2_ref, b2_ref, o_ref):
    x = x_ref[...]
    t = jnp.maximum(jnp.dot(x, w1_ref[...], preferred_element_type=jnp.float32)
                    + b1_ref[...], 0.0)
    y = jnp.dot(t, w2_ref[...], preferred_element_type=jnp.float32) + b2_ref[...]
    o_ref[...] = _ln(y)


def _enc_apply(x, p, block_rows):
    n, din = x.shape
    w1, b1 = p[0]["w"], p[0]["b"].reshape(1, -1)
    w2, b2 = p[1]["w"], p[1]["b"].reshape(1, -1)
    grid = (n // block_rows,)
    return pl.pallas_call(
        _enc_node_body,
        grid=grid,
        in_specs=[
            pl.BlockSpec((block_rows, din), lambda i: (i, 0)),
            pl.BlockSpec(w1.shape, lambda i: (0, 0)),
            pl.BlockSpec(b1.shape, lambda i: (0, 0)),
            pl.BlockSpec(w2.shape, lambda i: (0, 0)),
            pl.BlockSpec(b2.shape, lambda i: (0, 0)),
        ],
        out_specs=pl.BlockSpec((block_rows, D), lambda i: (i, 0)),
        out_shape=jax.ShapeDtypeStruct((n, D), jnp.float32),
    )(x, w1, b1, w2, b2)


def _prep_body(h_ref, ws_ref, wr_ref, a_ref, b_ref):
    h = h_ref[...]
    a_ref[...] = jnp.dot(h, ws_ref[...], preferred_element_type=jnp.float32)
    b_ref[...] = jnp.dot(h, wr_ref[...], preferred_element_type=jnp.float32)


def _prep_tables(h, ws, wr, block_rows=2000):
    grid = (N_NODES // block_rows,)
    return pl.pallas_call(
        _prep_body,
        grid=grid,
        in_specs=[
            pl.BlockSpec((block_rows, D), lambda i: (i, 0)),
            pl.BlockSpec((D, D), lambda i: (0, 0)),
            pl.BlockSpec((D, D), lambda i: (0, 0)),
        ],
        out_specs=[
            pl.BlockSpec((block_rows, D), lambda i: (i, 0)),
            pl.BlockSpec((block_rows, D), lambda i: (i, 0)),
        ],
        out_shape=[
            jax.ShapeDtypeStruct((N_NODES, D), jnp.float32),
            jax.ShapeDtypeStruct((N_NODES, D), jnp.float32),
        ],
    )(h, ws, wr)


def _edge_body(hs_ref, hr_ref, ee_ref, we_ref, b1_ref, w2_ref, b2_ref, o_ref):
    pre = (hs_ref[...] + hr_ref[...]
           + jnp.dot(ee_ref[...], we_ref[...], preferred_element_type=jnp.float32)
           + b1_ref[...])
    t = jnp.maximum(pre, 0.0)
    y = jnp.dot(t, w2_ref[...], preferred_element_type=jnp.float32) + b2_ref[...]
    o_ref[...] = _ln(y)


def _edge_mlp(hs, hr, ee, we, b1, w2, b2, block_rows=4000):
    grid = (N_EDGES // block_rows,)
    b1 = b1.reshape(1, -1)
    b2 = b2.reshape(1, -1)
    return pl.pallas_call(
        _edge_body,
        grid=grid,
        in_specs=[
            pl.BlockSpec((block_rows, D), lambda i: (i, 0)),
            pl.BlockSpec((block_rows, D), lambda i: (i, 0)),
            pl.BlockSpec((block_rows, D), lambda i: (i, 0)),
            pl.BlockSpec((D, D), lambda i: (0, 0)),
            pl.BlockSpec((1, D), lambda i: (0, 0)),
            pl.BlockSpec((D, D), lambda i: (0, 0)),
            pl.BlockSpec((1, D), lambda i: (0, 0)),
        ],
        out_specs=pl.BlockSpec((block_rows, D), lambda i: (i, 0)),
        out_shape=jax.ShapeDtypeStruct((N_EDGES, D), jnp.float32),
    )(hs, hr, ee, we, b1, w2, b2)


def _node_body(h_ref, p0_ref, p1_ref, u1h_ref, u1a_ref, b1_ref, u2_ref, b2_ref,
               o_ref):
    h = h_ref[...]
    agg = p0_ref[...] + p1_ref[...]
    t = jnp.maximum(
        jnp.dot(h, u1h_ref[...], preferred_element_type=jnp.float32)
        + jnp.dot(agg, u1a_ref[...], preferred_element_type=jnp.float32)
        + b1_ref[...], 0.0)
    y = jnp.dot(t, u2_ref[...], preferred_element_type=jnp.float32) + b2_ref[...]
    o_ref[...] = h + _ln(y)


def _node_mlp(h, p0, p1, u1h, u1a, b1, u2, b2, block_rows=2000):
    grid = (N_NODES // block_rows,)
    b1 = b1.reshape(1, -1)
    b2 = b2.reshape(1, -1)
    return pl.pallas_call(
        _node_body,
        grid=grid,
        in_specs=[
            pl.BlockSpec((block_rows, D), lambda i: (i, 0)),
            pl.BlockSpec((block_rows, D), lambda i: (i, 0)),
            pl.BlockSpec((block_rows, D), lambda i: (i, 0)),
            pl.BlockSpec((D, D), lambda i: (0, 0)),
            pl.BlockSpec((D, D), lambda i: (0, 0)),
            pl.BlockSpec((1, D), lambda i: (0, 0)),
            pl.BlockSpec((D, D), lambda i: (0, 0)),
            pl.BlockSpec((1, D), lambda i: (0, 0)),
        ],
        out_specs=pl.BlockSpec((block_rows, D), lambda i: (i, 0)),
        out_shape=jax.ShapeDtypeStruct((N_NODES, D), jnp.float32),
    )(h, p0, p1, u1h, u1a, b1, u2, b2)


_sc_mesh = plsc.VectorSubcoreMesh(core_axis_name="c", subcore_axis_name="s")


def _sc_gather(a_tab, b_tab, senders, receivers):
    @functools.partial(
        pl.kernel, mesh=_sc_mesh,
        out_type=[jax.ShapeDtypeStruct((N_EDGES, D), jnp.float32),
                  jax.ShapeDtypeStruct((N_EDGES, D), jnp.float32)],
        scratch_types=[
            pltpu.VMEM((_CH,), jnp.int32),
            pltpu.VMEM((_CH,), jnp.int32),
            pltpu.VMEM((_CH, D), jnp.float32),
            pltpu.VMEM((_CH, D), jnp.float32),
            pltpu.SemaphoreType.DMA,
            pltpu.SemaphoreType.DMA,
        ],
    )
    def k(a_hbm, b_hbm, s_hbm, r_hbm, hs_hbm, hr_hbm,
          sidx, ridx, abuf, bbuf, sema, semb):
        wid = lax.axis_index("s") * _NC + lax.axis_index("c")
        base = wid * _EPW
        last = _EPW - _CH

        @pl.loop(0, _FULL + 1)
        def _(j):
            off = base + jnp.minimum(j * _CH, last)
            pltpu.sync_copy(s_hbm.at[pl.ds(off, _CH)], sidx)
            pltpu.sync_copy(r_hbm.at[pl.ds(off, _CH)], ridx)
            cpa = pltpu.async_copy(a_hbm.at[sidx], abuf, sema)
            cpb = pltpu.async_copy(b_hbm.at[ridx], bbuf, semb)
            cpa.wait()
            cpb.wait()
            pltpu.sync_copy(abuf, hs_hbm.at[pl.ds(off, _CH)])
            pltpu.sync_copy(bbuf, hr_hbm.at[pl.ds(off, _CH)])

    return k(a_tab, b_tab, senders, receivers)


def _sc_scatter_add(msgs, receivers):
    @functools.partial(
        pl.kernel, mesh=_sc_mesh,
        out_type=jax.ShapeDtypeStruct((_NC, N_NODES, D), jnp.float32),
        scratch_types=[
            pltpu.VMEM((_CH,), jnp.int32),
            pltpu.VMEM((_TAIL,), jnp.int32),
            pltpu.VMEM((_CH, D), jnp.float32),
            pltpu.VMEM((_TAIL, D), jnp.float32),
            pltpu.VMEM_SHARED((N_NODES, D), jnp.float32),
        ],
    )
    def k(m_hbm, r_hbm, out_hbm, ridx, ridx_t, mbuf, mbuf_t, acc):
        c = lax.axis_index("c")
        s = lax.axis_index("s")

        @pl.loop(0, _TAIL)
        def _(i):
            @pl.loop(0, D, step=16)
            def _(q):
                mbuf_t.at[pl.ds(i, 1), pl.ds(q, 16)][...] = (
                    jnp.zeros((1, 16), jnp.float32))

        n_zchunks = N_NODES // _TAIL
        @pl.loop(0, (n_zchunks + _NS - 1) // _NS)
        def _(j):
            g = s + j * _NS
            @pl.when(g < n_zchunks)
            def _():
                pltpu.sync_copy(mbuf_t, acc.at[pl.ds(g * _TAIL, _TAIL)])

        plsc.subcore_barrier()

        base = (c * _NS + s) * _EPW
        @pl.loop(0, _FULL)
        def _(j):
            off = base + j * _CH
            pltpu.sync_copy(r_hbm.at[pl.ds(off, _CH)], ridx)
            pltpu.sync_copy(m_hbm.at[pl.ds(off, _CH)], mbuf)
            pltpu.sync_copy(mbuf, acc.at[ridx], add=True)

        off_t = base + _FULL * _CH
        pltpu.sync_copy(r_hbm.at[pl.ds(off_t, _TAIL)], ridx_t)
        pltpu.sync_copy(m_hbm.at[pl.ds(off_t, _TAIL)], mbuf_t)
        pltpu.sync_copy(mbuf_t, acc.at[ridx_t], add=True)

        plsc.subcore_barrier()

        @pl.loop(0, (n_zchunks + _NS - 1) // _NS)
        def _(j):
            g = s + j * _NS
            @pl.when(g < n_zchunks)
            def _():
                pltpu.sync_copy(acc.at[pl.ds(g * _TAIL, _TAIL)],
                                out_hbm.at[c].at[pl.ds(g * _TAIL, _TAIL)])

    return k(msgs, receivers)


def kernel(nodes, edges, senders, receivers, params):
    senders = senders.astype(jnp.int32)
    receivers = receivers.astype(jnp.int32)

    h = _enc_apply(nodes, params["enc_node"], block_rows=2000)
    ee = _enc_apply(edges, params["enc_edge"], block_rows=4000)

    for lp in params["layers"]:
        mw1 = lp["msg"][0]["w"]
        mb1 = lp["msg"][0]["b"]
        mw2, mb2 = lp["msg"][1]["w"], lp["msg"][1]["b"]
        ws, wr, we = mw1[:D], mw1[D:2 * D], mw1[2 * D:]

        a_tab, b_tab = _prep_tables(h, ws, wr)

        hs, hr = _sc_gather(a_tab, b_tab, senders, receivers)

        msgs = _edge_mlp(hs, hr, ee, we, mb1, mw2, mb2)

        partials = _sc_scatter_add(msgs, receivers)

        nw1 = lp["node"][0]["w"]
        nb1 = lp["node"][0]["b"]
        nw2, nb2 = lp["node"][1]["w"], lp["node"][1]["b"]
        h = _node_mlp(h, partials[0], partials[1], nw1[:D], nw1[D:],
                      nb1, nw2, nb2)

    return h


# speedup vs baseline: 5.0274x; 1.5128x over previous
import functools

import jax
import jax.numpy as jnp
from jax import lax
from jax.experimental import pallas as pl
from jax.experimental.pallas import tpu as pltpu
from jax.experimental.pallas import tpu_sc as plsc

N_NODES = 10000
N_EDGES = 320000
D = 128

_NC = 2
_NS = 16
_NW = _NC * _NS
_EPW = N_EDGES // _NW
_CH = 128
_FULL = _EPW // _CH
_TAIL = _EPW - _FULL * _CH


def _ln(x):
    mu = jnp.mean(x, axis=-1, keepdims=True)
    var = jnp.mean((x - mu) ** 2, axis=-1, keepdims=True)
    return (x - mu) / jnp.sqrt(var + 1e-6)


def _enc_node_body(x_ref, w1_ref, b1_ref, w2_ref, b2_ref, o_ref):
    x = x_ref[...]
    t = jnp.maximum(jnp.dot(x, w1_ref[...], preferred_element_type=jnp.float32)
                    + b1_ref[...], 0.0)
    y = jnp.dot(t, w2_ref[...], preferred_element_type=jnp.float32) + b2_ref[...]
    o_ref[...] = _ln(y)


def _enc_apply(x, p, block_rows):
    n, din = x.shape
    w1, b1 = p[0]["w"], p[0]["b"].reshape(1, -1)
    w2, b2 = p[1]["w"], p[1]["b"].reshape(1, -1)
    grid = (n // block_rows,)
    return pl.pallas_call(
        _enc_node_body,
        grid=grid,
        in_specs=[
            pl.BlockSpec((block_rows, din), lambda i: (i, 0)),
            pl.BlockSpec(w1.shape, lambda i: (0, 0)),
            pl.BlockSpec(b1.shape, lambda i: (0, 0)),
            pl.BlockSpec(w2.shape, lambda i: (0, 0)),
            pl.BlockSpec(b2.shape, lambda i: (0, 0)),
        ],
        out_specs=pl.BlockSpec((block_rows, D), lambda i: (i, 0)),
        out_shape=jax.ShapeDtypeStruct((n, D), jnp.float32),
    )(x, w1, b1, w2, b2)


def _prep_body(h_ref, ws_ref, wr_ref, a_ref, b_ref):
    h = h_ref[...]
    a_ref[...] = jnp.dot(h, ws_ref[...], preferred_element_type=jnp.float32)
    b_ref[...] = jnp.dot(h, wr_ref[...], preferred_element_type=jnp.float32)


def _prep_tables(h, ws, wr, block_rows=2000):
    grid = (N_NODES // block_rows,)
    return pl.pallas_call(
        _prep_body,
        grid=grid,
        in_specs=[
            pl.BlockSpec((block_rows, D), lambda i: (i, 0)),
            pl.BlockSpec((D, D), lambda i: (0, 0)),
            pl.BlockSpec((D, D), lambda i: (0, 0)),
        ],
        out_specs=[
            pl.BlockSpec((block_rows, D), lambda i: (i, 0)),
            pl.BlockSpec((block_rows, D), lambda i: (i, 0)),
        ],
        out_shape=[
            jax.ShapeDtypeStruct((N_NODES, D), jnp.float32),
            jax.ShapeDtypeStruct((N_NODES, D), jnp.float32),
        ],
    )(h, ws, wr)


def _edge_body(hs_ref, hr_ref, ee_ref, we_ref, b1_ref, w2_ref, b2_ref, o_ref):
    pre = (hs_ref[...] + hr_ref[...]
           + jnp.dot(ee_ref[...], we_ref[...], preferred_element_type=jnp.float32)
           + b1_ref[...])
    t = jnp.maximum(pre, 0.0)
    y = jnp.dot(t, w2_ref[...], preferred_element_type=jnp.float32) + b2_ref[...]
    o_ref[...] = _ln(y)


def _edge_mlp(hs, hr, ee, we, b1, w2, b2, block_rows=4000):
    grid = (N_EDGES // block_rows,)
    b1 = b1.reshape(1, -1)
    b2 = b2.reshape(1, -1)
    return pl.pallas_call(
        _edge_body,
        grid=grid,
        in_specs=[
            pl.BlockSpec((block_rows, D), lambda i: (i, 0)),
            pl.BlockSpec((block_rows, D), lambda i: (i, 0)),
            pl.BlockSpec((block_rows, D), lambda i: (i, 0)),
            pl.BlockSpec((D, D), lambda i: (0, 0)),
            pl.BlockSpec((1, D), lambda i: (0, 0)),
            pl.BlockSpec((D, D), lambda i: (0, 0)),
            pl.BlockSpec((1, D), lambda i: (0, 0)),
        ],
        out_specs=pl.BlockSpec((block_rows, D), lambda i: (i, 0)),
        out_shape=jax.ShapeDtypeStruct((N_EDGES, D), jnp.float32),
    )(hs, hr, ee, we, b1, w2, b2)


def _node_body(h_ref, p0_ref, p1_ref, u1h_ref, u1a_ref, b1_ref, u2_ref, b2_ref,
               o_ref):
    h = h_ref[...]
    agg = p0_ref[...] + p1_ref[...]
    t = jnp.maximum(
        jnp.dot(h, u1h_ref[...], preferred_element_type=jnp.float32)
        + jnp.dot(agg, u1a_ref[...], preferred_element_type=jnp.float32)
        + b1_ref[...], 0.0)
    y = jnp.dot(t, u2_ref[...], preferred_element_type=jnp.float32) + b2_ref[...]
    o_ref[...] = h + _ln(y)


def _node_mlp(h, p0, p1, u1h, u1a, b1, u2, b2, block_rows=2000):
    grid = (N_NODES // block_rows,)
    b1 = b1.reshape(1, -1)
    b2 = b2.reshape(1, -1)
    return pl.pallas_call(
        _node_body,
        grid=grid,
        in_specs=[
            pl.BlockSpec((block_rows, D), lambda i: (i, 0)),
            pl.BlockSpec((block_rows, D), lambda i: (i, 0)),
            pl.BlockSpec((block_rows, D), lambda i: (i, 0)),
            pl.BlockSpec((D, D), lambda i: (0, 0)),
            pl.BlockSpec((D, D), lambda i: (0, 0)),
            pl.BlockSpec((1, D), lambda i: (0, 0)),
            pl.BlockSpec((D, D), lambda i: (0, 0)),
            pl.BlockSpec((1, D), lambda i: (0, 0)),
        ],
        out_specs=pl.BlockSpec((block_rows, D), lambda i: (i, 0)),
        out_shape=jax.ShapeDtypeStruct((N_NODES, D), jnp.float32),
    )(h, p0, p1, u1h, u1a, b1, u2, b2)


_sc_mesh = plsc.VectorSubcoreMesh(core_axis_name="c", subcore_axis_name="s")


_GCH = 64
_SCH = 64
_NCHUNK_PAD = ((N_EDGES // _GCH + _NW - 1) // _NW) * _NW
_E_PAD = _NCHUNK_PAD * _GCH


def _sc_gather(a_tab, b_tab, s2d, r2d):
    @functools.partial(
        pl.kernel, mesh=_sc_mesh,
        out_type=[jax.ShapeDtypeStruct((_E_PAD, D), jnp.float32),
                  jax.ShapeDtypeStruct((_E_PAD, D), jnp.float32)],
        scratch_types=[
            pltpu.SemaphoreType.DMA,
            pltpu.SemaphoreType.DMA,
        ],
    )
    def k(a_hbm, b_hbm, s_hbm, r_hbm, hs_hbm, hr_hbm, sema, semb):
        def body(s_vmem, r_vmem, hs_vmem, hr_vmem):
            cpa = pltpu.async_copy(a_hbm.at[s_vmem.at[0]], hs_vmem, sema)
            cpb = pltpu.async_copy(b_hbm.at[r_vmem.at[0]], hr_vmem, semb)
            cpa.wait()
            cpb.wait()

        pltpu.emit_pipeline(
            body,
            grid=(_NCHUNK_PAD,),
            in_specs=[
                pl.BlockSpec((1, _GCH), index_map=lambda i: (i, 0)),
                pl.BlockSpec((1, _GCH), index_map=lambda i: (i, 0)),
            ],
            out_specs=[
                pl.BlockSpec((_GCH, D), index_map=lambda i: (i, 0)),
                pl.BlockSpec((_GCH, D), index_map=lambda i: (i, 0)),
            ],
            core_axis_name=("c", "s"),
            dimension_semantics=(pltpu.PARALLEL,),
        )(s_hbm, r_hbm, hs_hbm, hr_hbm)

    return k(a_tab, b_tab, s2d, r2d)


def _sc_scatter_add(msgs, receivers):
    n_chunks = N_EDGES // _SCH
    n_main = (n_chunks // _NW) * _NW
    _ZR = 16

    @functools.partial(
        pl.kernel, mesh=_sc_mesh,
        out_type=jax.ShapeDtypeStruct((_NC, N_NODES, D), jnp.float32),
        scratch_types=[
            pltpu.VMEM((_SCH,), jnp.int32),
            pltpu.VMEM((_SCH, D), jnp.float32),
            pltpu.VMEM((_ZR, D), jnp.float32),
            pltpu.VMEM_SHARED((N_NODES, D), jnp.float32),
        ],
    )
    def k(m_hbm, r_hbm, out_hbm, ridx_t, mbuf_t, zbuf, acc):
        c = lax.axis_index("c")
        s = lax.axis_index("s")
        wid = s * _NC + c

        @pl.loop(0, _ZR)
        def _(i):
            @pl.loop(0, D, step=16)
            def _(q):
                zbuf.at[pl.ds(i, 1), pl.ds(q, 16)][...] = (
                    jnp.zeros((1, 16), jnp.float32))

        n_zchunks = N_NODES // _ZR
        @pl.loop(0, (n_zchunks + _NS - 1) // _NS)
        def _(j):
            g = s + j * _NS
            @pl.when(g < n_zchunks)
            def _():
                pltpu.sync_copy(zbuf, acc.at[pl.ds(g * _ZR, _ZR)])

        plsc.subcore_barrier()

        def body(r_vmem, m_vmem):
            pltpu.sync_copy(m_vmem, acc.at[r_vmem.at[0]], add=True)

        pltpu.emit_pipeline(
            body,
            grid=(n_main,),
            in_specs=[
                pl.BlockSpec((1, _SCH), index_map=lambda i: (i, 0)),
                pl.BlockSpec((_SCH, D), index_map=lambda i: (i, 0)),
            ],
            out_specs=[],
            core_axis_name=("c", "s"),
            dimension_semantics=(pltpu.PARALLEL,),
        )(r_hbm, m_hbm)

        @pl.when(wid < n_chunks - n_main)
        def _():
            tc = n_main + wid
            pltpu.sync_copy(r_hbm.at[tc], ridx_t)
            pltpu.sync_copy(m_hbm.at[pl.ds(tc * _SCH, _SCH)], mbuf_t)
            pltpu.sync_copy(mbuf_t, acc.at[ridx_t], add=True)

        plsc.subcore_barrier()

        @pl.loop(0, (n_zchunks + _NS - 1) // _NS)
        def _(j):
            g = s + j * _NS
            @pl.when(g < n_zchunks)
            def _():
                pltpu.sync_copy(acc.at[pl.ds(g * _ZR, _ZR)],
                                out_hbm.at[c].at[pl.ds(g * _ZR, _ZR)])

    return k(msgs, receivers)


def kernel(nodes, edges, senders, receivers, params):
    senders = senders.astype(jnp.int32)
    receivers = receivers.astype(jnp.int32)
    pad = _E_PAD - N_EDGES
    s2d = jnp.pad(senders, (0, pad)).reshape(_NCHUNK_PAD, _GCH)
    r2d_pad = jnp.pad(receivers, (0, pad)).reshape(_NCHUNK_PAD, _GCH)
    r2d = receivers.reshape(N_EDGES // _SCH, _SCH)

    h = _enc_apply(nodes, params["enc_node"], block_rows=2000)
    ee = _enc_apply(edges, params["enc_edge"], block_rows=4000)

    for lp in params["layers"]:
        mw1 = lp["msg"][0]["w"]
        mb1 = lp["msg"][0]["b"]
        mw2, mb2 = lp["msg"][1]["w"], lp["msg"][1]["b"]
        ws, wr, we = mw1[:D], mw1[D:2 * D], mw1[2 * D:]

        a_tab, b_tab = _prep_tables(h, ws, wr)

        hs, hr = _sc_gather(a_tab, b_tab, s2d, r2d_pad)

        msgs = _edge_mlp(hs, hr, ee, we, mb1, mw2, mb2)

        partials = _sc_scatter_add(msgs, r2d)

        nw1 = lp["node"][0]["w"]
        nb1 = lp["node"][0]["b"]
        nw2, nb2 = lp["node"][1]["w"], lp["node"][1]["b"]
        h = _node_mlp(h, partials[0], partials[1], nw1[:D], nw1[D:],
                      nb1, nw2, nb2)

    return h
